# baseline (speedup 1.0000x reference)
import functools
import jax
import jax.numpy as jnp
from jax import lax
from jax.experimental import pallas as pl
from jax.experimental.pallas import tpu as pltpu
from jax.experimental.pallas import tpu_sc as plsc

N = 10000
E = 320000
D = 128
D_OUT = 40

NC = 2
NS = 16
NW = NC * NS
EPW = E // NW
CHUNK = 80
EPW_PAD = 10240
NCHUNK = EPW_PAD // CHUNK
NPAD = 10240
RPT = NPAD // NS
WCHUNK = 128


@functools.lru_cache(maxsize=None)
def _sc_agg(with_deg: bool):
  mesh = plsc.VectorSubcoreMesh(core_axis_name="c", subcore_axis_name="s",
                                num_cores=NC, num_subcores=NS)

  out_type = jax.ShapeDtypeStruct((NC, NPAD, D), jnp.float32)
  scratch = [
      pltpu.VMEM((CHUNK,), jnp.int32),
      pltpu.VMEM((CHUNK,), jnp.int32),
      pltpu.VMEM((CHUNK, D), jnp.float32),
      pltpu.VMEM((WCHUNK, D), jnp.float32),
      pltpu.VMEM_SHARED((NPAD, D), jnp.float32),
      pltpu.SemaphoreType.DMA,
  ]
  DR = NPAD // D
  if with_deg:
    out_type = [out_type, jax.ShapeDtypeStruct((NC, DR, D), jnp.float32)]
    scratch.append(pltpu.VMEM((NPAD,), jnp.float32))
    scratch.append(pltpu.VMEM((DR, D), jnp.float32))
    scratch.append(pltpu.VMEM_SHARED((DR, D), jnp.float32))
    scratch.append(pltpu.VMEM((DR,), jnp.int32))
  else:
    scratch.append(pltpu.VMEM((CHUNK,), jnp.int32))
    scratch.append(pltpu.VMEM((CHUNK,), jnp.int32))
    scratch.append(pltpu.VMEM((CHUNK, D), jnp.float32))
    scratch.append(pltpu.SemaphoreType.DMA)
    scratch.append(pltpu.SemaphoreType.DMA)
    scratch.append(pltpu.SemaphoreType.DMA)

  @functools.partial(
      pl.kernel, out_type=out_type, mesh=mesh, scratch_types=scratch,
      compiler_params=pltpu.CompilerParams(needs_layout_passes=False))
  def agg(h_hbm, src_hbm, dst_hbm, out_hbm, *rest):
    if with_deg:
      (deg_hbm, src_v, dst_v, rows_v, buf_v, acc_sh, sem, deg_v,
       deg2_v, deg_sh, iota_v) = rest
    else:
      src_v, dst_v, rows_v, buf_v, acc_sh, sem, *_unused = rest
    c = lax.axis_index("c")
    s = lax.axis_index("s")
    wid = s * NC + c

    zeros16 = jnp.zeros((16,), jnp.float32)
    ones16 = jnp.ones((16,), jnp.float32)

    def zrow(r, _):
      for j in range(D // 16):
        buf_v[r, pl.ds(j * 16, 16)] = zeros16
      return 0

    lax.fori_loop(0, WCHUNK, zrow, 0)
    row0 = s * RPT
    for k in range(RPT // WCHUNK):
      pltpu.sync_copy(buf_v, acc_sh.at[pl.ds(row0 + k * WCHUNK, WCHUNK)])
    if with_deg:
      def zdeg(i, _):
        deg_v[pl.ds(i * 16, 16)] = zeros16
        return 0
      lax.fori_loop(0, NPAD // 16, zdeg, 0)
      iota16 = lax.iota(jnp.int32, 16)
      for i in range(DR // 16):
        iota_v[pl.ds(i * 16, 16)] = iota16 + (i * 16)
      @pl.when(s < DR // 8)
      def _():
        pltpu.sync_copy(buf_v.at[pl.ds(0, 8)], deg_sh.at[pl.ds(s * 8, 8)])
    plsc.subcore_barrier()

    def step(t, _):
      base = wid * EPW_PAD + t * CHUNK
      pltpu.sync_copy(src_hbm.at[pl.ds(base, CHUNK)], src_v)
      pltpu.sync_copy(dst_hbm.at[pl.ds(base, CHUNK)], dst_v)
      pltpu.async_copy(h_hbm.at[src_v], rows_v, sem).wait()
      if with_deg:
        for j in range(CHUNK // 16):
          idx = dst_v[pl.ds(j * 16, 16)]
          plsc.addupdate_scatter(deg_v, [idx], ones16)
      pltpu.sync_copy(rows_v, acc_sh.at[dst_v], add=True)
      return 0

    lax.fori_loop(0, NCHUNK, step, 0)
    plsc.subcore_barrier()

    if with_deg:
      def stage(r, _):
        for j in range(D // 16):
          deg2_v[r, pl.ds(j * 16, 16)] = deg_v[pl.ds(r * D + j * 16, 16)]
        return 0
      lax.fori_loop(0, DR, stage, 0)
      pltpu.sync_copy(deg2_v, deg_sh.at[iota_v], add=True)
      plsc.subcore_barrier()

    for k in range(RPT // WCHUNK):
      r0 = row0 + k * WCHUNK
      pltpu.sync_copy(acc_sh.at[pl.ds(r0, WCHUNK)], buf_v)
      pltpu.sync_copy(buf_v, out_hbm.at[c, pl.ds(r0, WCHUNK)])
    if with_deg:
      @pl.when(s < DR // 8)
      def _():
        pltpu.sync_copy(deg_sh.at[pl.ds(s * 8, 8)], deg2_v.at[pl.ds(0, 8)])
        pltpu.sync_copy(deg2_v.at[pl.ds(0, 8)],
                        deg_hbm.at[c, pl.ds(s * 8, 8)])

  return agg


BR = 2000


def _make_comb_body(relu: bool):
  def body(h_ref, s0_ref, s1_ref, d0_ref, d1_ref, ws_ref, wn_ref, b_ref,
           o_ref):
    inv = 1.0 / jnp.maximum(d0_ref[...] + d1_ref[...], 1.0)
    hn = (s0_ref[...] + s1_ref[...]) * inv
    h = h_ref[...]
    o = (jnp.dot(h, ws_ref[...], preferred_element_type=jnp.float32)
         + jnp.dot(hn, wn_ref[...], preferred_element_type=jnp.float32)
         + b_ref[...])
    o_ref[...] = jnp.maximum(o, 0.0) if relu else o
  return body


def _tc_comb(h, s0, s1, d0, d1, ws, wn, b, relu):
  n, d = h.shape
  do = ws.shape[1]
  return pl.pallas_call(
      _make_comb_body(relu),
      grid=(n // BR,),
      in_specs=[
          pl.BlockSpec((BR, d), lambda i: (i, 0)),
          pl.BlockSpec((BR, d), lambda i: (i, 0)),
          pl.BlockSpec((BR, d), lambda i: (i, 0)),
          pl.BlockSpec((BR, 1), lambda i: (i, 0)),
          pl.BlockSpec((BR, 1), lambda i: (i, 0)),
          pl.BlockSpec((d, do), lambda i: (0, 0)),
          pl.BlockSpec((d, do), lambda i: (0, 0)),
          pl.BlockSpec((1, do), lambda i: (0, 0)),
      ],
      out_specs=pl.BlockSpec((BR, do), lambda i: (i, 0)),
      out_shape=jax.ShapeDtypeStruct((n, do), jnp.float32),
  )(h, s0, s1, d0, d1, ws, wn, b)


def kernel(x, edge_index, edge_weight,
           W_self1, W_neigh1, b1,
           W_self2, W_neigh2, b2,
           W_self3, W_neigh3, b3):
  src = edge_index[0].astype(jnp.int32)
  dst = edge_index[1].astype(jnp.int32)

  npe = EPW_PAD - EPW
  pad_dst = N + jnp.arange(npe, dtype=jnp.int32) % (NPAD - N)
  src_f = jnp.pad(src.reshape(NW, EPW), ((0, 0), (0, npe))).reshape(-1)
  dst_f = jnp.concatenate(
      [dst.reshape(NW, EPW), jnp.broadcast_to(pad_dst, (NW, npe))],
      axis=1).reshape(-1)
  src, dst = src_f, dst_f

  p1, degp = _sc_agg(True)(x, src, dst)
  degf = degp.reshape(NC, NPAD)
  d0 = degf[0, :N].reshape(N, 1)
  d1 = degf[1, :N].reshape(N, 1)

  h1 = _tc_comb(x, p1[0, :N], p1[1, :N], d0, d1,
                W_self1, W_neigh1, b1.reshape(1, -1), relu=True)
  p2 = _sc_agg(False)(h1, src, dst)
  h2 = _tc_comb(h1, p2[0, :N], p2[1, :N], d0, d1,
                W_self2, W_neigh2, b2.reshape(1, -1), relu=True)
  p3 = _sc_agg(False)(h2, src, dst)
  out = _tc_comb(h2, p3[0, :N], p3[1, :N], d0, d1,
                 W_self3, W_neigh3, b3.reshape(1, -1), relu=False)
  return out


# speedup vs baseline: 3.4908x; 3.4908x over previous
import functools
import jax
import jax.numpy as jnp
from jax import lax
from jax.experimental import pallas as pl
from jax.experimental.pallas import tpu as pltpu
from jax.experimental.pallas import tpu_sc as plsc

N = 10000
E = 320000
D = 128
D_OUT = 40

NC = 2
NS = 16
NW = NC * NS
EPW = E // NW
CHUNK = 80
NCHUNK = EPW // CHUNK
NPAD = 10240
RPT = NPAD // NS
WCHUNK = 80
DR = NPAD // D


@functools.lru_cache(maxsize=None)
def _sc_agg(with_deg: bool):
  mesh = plsc.VectorSubcoreMesh(core_axis_name="c", subcore_axis_name="s",
                                num_cores=NC, num_subcores=NS)

  out_type = jax.ShapeDtypeStruct((NC, NPAD, D), jnp.float32)
  scratch = [
      pltpu.VMEM((CHUNK,), jnp.int32),
      pltpu.VMEM((CHUNK,), jnp.int32),
      pltpu.VMEM((CHUNK,), jnp.int32),
      pltpu.VMEM((CHUNK,), jnp.int32),
      pltpu.VMEM((CHUNK, D), jnp.float32),
      pltpu.VMEM((CHUNK, D), jnp.float32),
      pltpu.VMEM_SHARED((NPAD, D), jnp.float32),
      pltpu.SemaphoreType.DMA,
      pltpu.SemaphoreType.DMA,
      pltpu.SemaphoreType.DMA,
      pltpu.SemaphoreType.DMA,
  ]
  if with_deg:
    out_type = [out_type, jax.ShapeDtypeStruct((NC, DR, D), jnp.float32)]
    scratch.append(pltpu.VMEM((NPAD,), jnp.float32))
    scratch.append(pltpu.VMEM((DR, D), jnp.float32))
    scratch.append(pltpu.VMEM_SHARED((DR, D), jnp.float32))
    scratch.append(pltpu.VMEM((DR,), jnp.int32))

  @functools.partial(
      pl.kernel, out_type=out_type, mesh=mesh, scratch_types=scratch,
      compiler_params=pltpu.CompilerParams(needs_layout_passes=False))
  def agg(h_hbm, src_hbm, dst_hbm, out_hbm, *rest):
    if with_deg:
      (deg_hbm, src0_v, src1_v, dst0_v, dst1_v, rows0_v, rows1_v, acc_sh,
       semr0, semr1, semi0, semi1, deg_v, deg2_v, deg_sh, iota_v) = rest
    else:
      (src0_v, src1_v, dst0_v, dst1_v, rows0_v, rows1_v, acc_sh,
       semr0, semr1, semi0, semi1) = rest
    c = lax.axis_index("c")
    s = lax.axis_index("s")
    wid = s * NC + c
    base = wid * EPW

    zeros16 = jnp.zeros((16,), jnp.float32)
    ones16 = jnp.ones((16,), jnp.float32)

    def load_idx(t, sv, dv, semi):
      pltpu.async_copy(src_hbm.at[pl.ds(base + t * CHUNK, CHUNK)], sv, semi)
      pltpu.async_copy(dst_hbm.at[pl.ds(base + t * CHUNK, CHUNK)], dv, semi)

    def wait_idx(sv, dv, semi):
      pltpu.make_async_copy(src_hbm.at[pl.ds(base, CHUNK)], sv, semi).wait()
      pltpu.make_async_copy(dst_hbm.at[pl.ds(base, CHUNK)], dv, semi).wait()

    def gather(sv, rows, semr):
      pltpu.async_copy(h_hbm.at[sv], rows, semr)

    def drain(rows, semr):
      pltpu.make_async_copy(h_hbm.at[src0_v], rows, semr).wait()

    def scatter(dv, rows):
      if with_deg:
        for j in range(CHUNK // 16):
          idx = dv[pl.ds(j * 16, 16)]
          plsc.addupdate_scatter(deg_v, [idx], ones16)
      pltpu.sync_copy(rows, acc_sh.at[dv], add=True)

    load_idx(0, src0_v, dst0_v, semi0)

    def zrow(r, _):
      for j in range(D // 16):
        rows0_v[r, pl.ds(j * 16, 16)] = zeros16
      return 0

    lax.fori_loop(0, WCHUNK, zrow, 0)
    row0 = s * RPT
    for k in range(RPT // WCHUNK):
      pltpu.sync_copy(rows0_v, acc_sh.at[pl.ds(row0 + k * WCHUNK, WCHUNK)])
    if with_deg:
      def zdeg(i, _):
        deg_v[pl.ds(i * 16, 16)] = zeros16
        return 0
      lax.fori_loop(0, NPAD // 16, zdeg, 0)
      iota16 = lax.iota(jnp.int32, 16)
      for i in range(DR // 16):
        iota_v[pl.ds(i * 16, 16)] = iota16 + (i * 16)
      @pl.when(s < DR // 8)
      def _():
        pltpu.sync_copy(rows0_v.at[pl.ds(0, 8)], deg_sh.at[pl.ds(s * 8, 8)])
    plsc.subcore_barrier()

    wait_idx(src0_v, dst0_v, semi0)
    gather(src0_v, rows0_v, semr0)
    load_idx(1, src1_v, dst1_v, semi1)

    def phase(t0, sA, dA, rA, semrA, semiA, sB, dB, rB, semrB, semiB):
      wait_idx(sB, dB, semiB)
      gather(sB, rB, semrB)
      drain(rA, semrA)
      scatter(dA, rA)
      load_idx(t0 + 2, sA, dA, semiA)

    def pair(p, _):
      t0 = p * 2
      phase(t0, src0_v, dst0_v, rows0_v, semr0, semi0,
            src1_v, dst1_v, rows1_v, semr1, semi1)
      phase(t0 + 1, src1_v, dst1_v, rows1_v, semr1, semi1,
            src0_v, dst0_v, rows0_v, semr0, semi0)
      return 0

    lax.fori_loop(0, (NCHUNK - 3) // 2, pair, 0)
    wait_idx(src1_v, dst1_v, semi1)
    gather(src1_v, rows1_v, semr1)
    drain(rows0_v, semr0)
    scatter(dst0_v, rows0_v)
    load_idx(NCHUNK - 1, src0_v, dst0_v, semi0)
    wait_idx(src0_v, dst0_v, semi0)
    gather(src0_v, rows0_v, semr0)
    drain(rows1_v, semr1)
    scatter(dst1_v, rows1_v)
    drain(rows0_v, semr0)
    scatter(dst0_v, rows0_v)
    plsc.subcore_barrier()

    if with_deg:
      def stage(r, _):
        for j in range(D // 16):
          deg2_v[r, pl.ds(j * 16, 16)] = deg_v[pl.ds(r * D + j * 16, 16)]
        return 0
      lax.fori_loop(0, DR, stage, 0)
      pltpu.sync_copy(deg2_v, deg_sh.at[iota_v], add=True)
      plsc.subcore_barrier()

    for k in range(RPT // WCHUNK):
      r0 = row0 + k * WCHUNK
      pltpu.sync_copy(acc_sh.at[pl.ds(r0, WCHUNK)], rows0_v)
      pltpu.sync_copy(rows0_v, out_hbm.at[c, pl.ds(r0, WCHUNK)])
    if with_deg:
      @pl.when(s < DR // 8)
      def _():
        pltpu.sync_copy(deg_sh.at[pl.ds(s * 8, 8)], deg2_v.at[pl.ds(0, 8)])
        pltpu.sync_copy(deg2_v.at[pl.ds(0, 8)],
                        deg_hbm.at[c, pl.ds(s * 8, 8)])

  return agg


BR = 2000


def _make_comb_body(relu: bool):
  def body(h_ref, s0_ref, s1_ref, d0_ref, d1_ref, ws_ref, wn_ref, b_ref,
           o_ref):
    inv = 1.0 / jnp.maximum(d0_ref[...] + d1_ref[...], 1.0)
    hn = (s0_ref[...] + s1_ref[...]) * inv
    h = h_ref[...]
    o = (jnp.dot(h, ws_ref[...], preferred_element_type=jnp.float32)
         + jnp.dot(hn, wn_ref[...], preferred_element_type=jnp.float32)
         + b_ref[...])
    o_ref[...] = jnp.maximum(o, 0.0) if relu else o
  return body


def _tc_comb(h, s0, s1, d0, d1, ws, wn, b, relu):
  n, d = h.shape
  do = ws.shape[1]
  return pl.pallas_call(
      _make_comb_body(relu),
      grid=(n // BR,),
      in_specs=[
          pl.BlockSpec((BR, d), lambda i: (i, 0)),
          pl.BlockSpec((BR, d), lambda i: (i, 0)),
          pl.BlockSpec((BR, d), lambda i: (i, 0)),
          pl.BlockSpec((BR, 1), lambda i: (i, 0)),
          pl.BlockSpec((BR, 1), lambda i: (i, 0)),
          pl.BlockSpec((d, do), lambda i: (0, 0)),
          pl.BlockSpec((d, do), lambda i: (0, 0)),
          pl.BlockSpec((1, do), lambda i: (0, 0)),
      ],
      out_specs=pl.BlockSpec((BR, do), lambda i: (i, 0)),
      out_shape=jax.ShapeDtypeStruct((n, do), jnp.float32),
  )(h, s0, s1, d0, d1, ws, wn, b)


def kernel(x, edge_index, edge_weight,
           W_self1, W_neigh1, b1,
           W_self2, W_neigh2, b2,
           W_self3, W_neigh3, b3):
  src = edge_index[0].astype(jnp.int32)
  dst = edge_index[1].astype(jnp.int32)

  p1, degp = _sc_agg(True)(x, src, dst)
  degf = degp.reshape(NC, NPAD)
  d0 = degf[0, :N].reshape(N, 1)
  d1 = degf[1, :N].reshape(N, 1)

  h1 = _tc_comb(x, p1[0, :N], p1[1, :N], d0, d1,
                W_self1, W_neigh1, b1.reshape(1, -1), relu=True)
  p2 = _sc_agg(False)(h1, src, dst)
  h2 = _tc_comb(h1, p2[0, :N], p2[1, :N], d0, d1,
                W_self2, W_neigh2, b2.reshape(1, -1), relu=True)
  p3 = _sc_agg(False)(h2, src, dst)
  out = _tc_comb(h2, p3[0, :N], p3[1, :N], d0, d1,
                 W_self3, W_neigh3, b3.reshape(1, -1), relu=False)
  return out


# speedup vs baseline: 3.6965x; 1.0589x over previous
import functools
import jax
import jax.numpy as jnp
from jax import lax
from jax.experimental import pallas as pl
from jax.experimental.pallas import tpu as pltpu
from jax.experimental.pallas import tpu_sc as plsc

N = 10000
E = 320000
D = 128
D_OUT = 40

NC = 2
NS = 16
NW = NC * NS
EPW = E // NW
CHUNK = 80
NCHUNK = EPW // CHUNK
NPAD = 10240
RPT = NPAD // NS
WCHUNK = 80
DR = NPAD // D


@functools.lru_cache(maxsize=None)
def _sc_agg(with_deg: bool):
  mesh = plsc.VectorSubcoreMesh(core_axis_name="c", subcore_axis_name="s",
                                num_cores=NC, num_subcores=NS)

  out_type = jax.ShapeDtypeStruct((NC, NPAD, D), jnp.float32)
  scratch = [
      pltpu.VMEM((CHUNK,), jnp.int32),
      pltpu.VMEM((CHUNK,), jnp.int32),
      pltpu.VMEM((CHUNK,), jnp.int32),
      pltpu.VMEM((CHUNK,), jnp.int32),
      pltpu.VMEM((CHUNK,), jnp.int32),
      pltpu.VMEM((CHUNK,), jnp.int32),
      pltpu.VMEM((CHUNK, D), jnp.float32),
      pltpu.VMEM((CHUNK, D), jnp.float32),
      pltpu.VMEM((CHUNK, D), jnp.float32),
      pltpu.VMEM_SHARED((NPAD, D), jnp.float32),
      pltpu.SemaphoreType.DMA,
      pltpu.SemaphoreType.DMA,
      pltpu.SemaphoreType.DMA,
      pltpu.SemaphoreType.DMA,
      pltpu.SemaphoreType.DMA,
      pltpu.SemaphoreType.DMA,
  ]
  if with_deg:
    out_type = [out_type, jax.ShapeDtypeStruct((NC, DR, D), jnp.float32)]
    scratch.append(pltpu.VMEM((NPAD,), jnp.float32))
    scratch.append(pltpu.VMEM_SHARED((DR, D), jnp.float32))
    scratch.append(pltpu.VMEM((DR,), jnp.int32))

  @functools.partial(
      pl.kernel, out_type=out_type, mesh=mesh, scratch_types=scratch,
      compiler_params=pltpu.CompilerParams(needs_layout_passes=False))
  def agg(h_hbm, src_hbm, dst_hbm, out_hbm, *rest):
    if with_deg:
      (deg_hbm, src0_v, src1_v, src2_v, dst0_v, dst1_v, dst2_v,
       rows0_v, rows1_v, rows2_v, acc_sh,
       semr0, semr1, semr2, semi0, semi1, semi2,
       deg_v, deg_sh, iota_v) = rest
    else:
      (src0_v, src1_v, src2_v, dst0_v, dst1_v, dst2_v,
       rows0_v, rows1_v, rows2_v, acc_sh,
       semr0, semr1, semr2, semi0, semi1, semi2) = rest
    c = lax.axis_index("c")
    s = lax.axis_index("s")
    wid = s * NC + c
    base = wid * EPW

    zeros16 = jnp.zeros((16,), jnp.float32)
    ones16 = jnp.ones((16,), jnp.float32)

    def load_idx(t, sv, dv, semi):
      pltpu.async_copy(src_hbm.at[pl.ds(base + t * CHUNK, CHUNK)], sv, semi)
      pltpu.async_copy(dst_hbm.at[pl.ds(base + t * CHUNK, CHUNK)], dv, semi)

    def wait_idx(sv, dv, semi):
      pltpu.make_async_copy(src_hbm.at[pl.ds(base, CHUNK)], sv, semi).wait()
      pltpu.make_async_copy(dst_hbm.at[pl.ds(base, CHUNK)], dv, semi).wait()

    def gather(sv, rows, semr):
      pltpu.async_copy(h_hbm.at[sv], rows, semr)

    def drain(rows, semr):
      pltpu.make_async_copy(h_hbm.at[src0_v], rows, semr).wait()

    def scatter(dv, rows):
      if with_deg:
        for j in range(CHUNK // 16):
          idx = dv[pl.ds(j * 16, 16)]
          plsc.addupdate_scatter(deg_v, [idx], ones16)
      pltpu.sync_copy(rows, acc_sh.at[dv], add=True)

    load_idx(0, src0_v, dst0_v, semi0)

    def zrow(r, _):
      for j in range(D // 16):
        rows0_v[r, pl.ds(j * 16, 16)] = zeros16
      return 0

    lax.fori_loop(0, WCHUNK, zrow, 0)
    row0 = s * RPT
    for k in range(RPT // WCHUNK):
      pltpu.sync_copy(rows0_v, acc_sh.at[pl.ds(row0 + k * WCHUNK, WCHUNK)])
    if with_deg:
      def zdeg(i, _):
        deg_v[pl.ds(i * 16, 16)] = zeros16
        return 0
      lax.fori_loop(0, NPAD // 16, zdeg, 0)
      iota16 = lax.iota(jnp.int32, 16)
      for i in range(DR // 16):
        iota_v[pl.ds(i * 16, 16)] = iota16 + (i * 16)
      @pl.when(s < DR // 8)
      def _():
        pltpu.sync_copy(rows0_v.at[pl.ds(0, 8)], deg_sh.at[pl.ds(s * 8, 8)])
    plsc.subcore_barrier()

    slot0 = (src0_v, dst0_v, rows0_v, semr0, semi0)
    slot1 = (src1_v, dst1_v, rows1_v, semr1, semi1)
    slot2 = (src2_v, dst2_v, rows2_v, semr2, semi2)

    wait_idx(src0_v, dst0_v, semi0)
    gather(src0_v, rows0_v, semr0)
    load_idx(1, src1_v, dst1_v, semi1)
    wait_idx(src1_v, dst1_v, semi1)
    gather(src1_v, rows1_v, semr1)
    load_idx(2, src2_v, dst2_v, semi2)

    def phase(t, A, B, C):
      del B
      sC, dC, rC, semrC, semiC = C
      sA, dA, rA, semrA, semiA = A
      wait_idx(sC, dC, semiC)
      gather(sC, rC, semrC)
      drain(rA, semrA)
      scatter(dA, rA)
      load_idx(t + 3, sA, dA, semiA)

    def triple(p, _):
      t0 = p * 3
      phase(t0, slot0, slot1, slot2)
      phase(t0 + 1, slot1, slot2, slot0)
      phase(t0 + 2, slot2, slot0, slot1)
      return 0

    lax.fori_loop(0, (NCHUNK - 5) // 3, triple, 0)
    wait_idx(src2_v, dst2_v, semi2)
    gather(src2_v, rows2_v, semr2)
    drain(rows0_v, semr0)
    scatter(dst0_v, rows0_v)
    load_idx(NCHUNK - 2, src0_v, dst0_v, semi0)
    wait_idx(src0_v, dst0_v, semi0)
    gather(src0_v, rows0_v, semr0)
    drain(rows1_v, semr1)
    scatter(dst1_v, rows1_v)
    load_idx(NCHUNK - 1, src1_v, dst1_v, semi1)
    wait_idx(src1_v, dst1_v, semi1)
    gather(src1_v, rows1_v, semr1)
    drain(rows2_v, semr2)
    scatter(dst2_v, rows2_v)
    drain(rows0_v, semr0)
    scatter(dst0_v, rows0_v)
    drain(rows1_v, semr1)
    scatter(dst1_v, rows1_v)
    plsc.subcore_barrier()

    if with_deg:
      def stage(r, _):
        for j in range(D // 16):
          rows1_v[r, pl.ds(j * 16, 16)] = deg_v[pl.ds(r * D + j * 16, 16)]
        return 0
      lax.fori_loop(0, DR, stage, 0)
      pltpu.sync_copy(rows1_v, deg_sh.at[iota_v], add=True)
      plsc.subcore_barrier()

    for k in range(RPT // WCHUNK):
      r0 = row0 + k * WCHUNK
      pltpu.sync_copy(acc_sh.at[pl.ds(r0, WCHUNK)], rows0_v)
      pltpu.sync_copy(rows0_v, out_hbm.at[c, pl.ds(r0, WCHUNK)])
    if with_deg:
      @pl.when(s < DR // 8)
      def _():
        pltpu.sync_copy(deg_sh.at[pl.ds(s * 8, 8)], rows1_v.at[pl.ds(0, 8)])
        pltpu.sync_copy(rows1_v.at[pl.ds(0, 8)],
                        deg_hbm.at[c, pl.ds(s * 8, 8)])

  return agg


BR = 2000


def _make_comb_body(relu: bool):
  def body(h_ref, s0_ref, s1_ref, d0_ref, d1_ref, ws_ref, wn_ref, b_ref,
           o_ref):
    inv = 1.0 / jnp.maximum(d0_ref[...] + d1_ref[...], 1.0)
    hn = (s0_ref[...] + s1_ref[...]) * inv
    h = h_ref[...]
    o = (jnp.dot(h, ws_ref[...], preferred_element_type=jnp.float32)
         + jnp.dot(hn, wn_ref[...], preferred_element_type=jnp.float32)
         + b_ref[...])
    o_ref[...] = jnp.maximum(o, 0.0) if relu else o
  return body


def _tc_comb(h, s0, s1, d0, d1, ws, wn, b, relu):
  n, d = h.shape
  do = ws.shape[1]
  return pl.pallas_call(
      _make_comb_body(relu),
      grid=(n // BR,),
      in_specs=[
          pl.BlockSpec((BR, d), lambda i: (i, 0)),
          pl.BlockSpec((BR, d), lambda i: (i, 0)),
          pl.BlockSpec((BR, d), lambda i: (i, 0)),
          pl.BlockSpec((BR, 1), lambda i: (i, 0)),
          pl.BlockSpec((BR, 1), lambda i: (i, 0)),
          pl.BlockSpec((d, do), lambda i: (0, 0)),
          pl.BlockSpec((d, do), lambda i: (0, 0)),
          pl.BlockSpec((1, do), lambda i: (0, 0)),
      ],
      out_specs=pl.BlockSpec((BR, do), lambda i: (i, 0)),
      out_shape=jax.ShapeDtypeStruct((n, do), jnp.float32),
  )(h, s0, s1, d0, d1, ws, wn, b)


def kernel(x, edge_index, edge_weight,
           W_self1, W_neigh1, b1,
           W_self2, W_neigh2, b2,
           W_self3, W_neigh3, b3):
  src = edge_index[0].astype(jnp.int32)
  dst = edge_index[1].astype(jnp.int32)

  p1, degp = _sc_agg(True)(x, src, dst)
  degf = degp.reshape(NC, NPAD)
  d0 = degf[0, :N].reshape(N, 1)
  d1 = degf[1, :N].reshape(N, 1)

  h1 = _tc_comb(x, p1[0, :N], p1[1, :N], d0, d1,
                W_self1, W_neigh1, b1.reshape(1, -1), relu=True)
  p2 = _sc_agg(False)(h1, src, dst)
  h2 = _tc_comb(h1, p2[0, :N], p2[1, :N], d0, d1,
                W_self2, W_neigh2, b2.reshape(1, -1), relu=True)
  p3 = _sc_agg(False)(h2, src, dst)
  out = _tc_comb(h2, p3[0, :N], p3[1, :N], d0, d1,
                 W_self3, W_neigh3, b3.reshape(1, -1), relu=False)
  return out
